# baseline (speedup 1.0000x reference)
import functools

import jax
import jax.numpy as jnp
from jax import lax
from jax.experimental import pallas as pl
from jax.experimental.pallas import tpu as pltpu

F32 = jnp.float32
BF16 = jnp.bfloat16


def _cparams():
    return pltpu.CompilerParams(dimension_semantics=("parallel",),
                                vmem_limit_bytes=56 * 1024 * 1024)


def _row_tile(n):
    for t in (4096, 2048, 1024, 512, 256, 128, 64, 32, 16, 8):
        if n % t == 0:
            return t
    return n


def _relu(v):
    return jnp.maximum(v, 0.0)


def _dotf(a, b):
    return jnp.dot(a, b, preferred_element_type=F32)


def _dotb(a, w):
    return jnp.dot(a.astype(BF16), w, preferred_element_type=F32)


def _encode_pj_kernel(x_ref, pos_ref, we1, be1, we2, be2, wt, bt,
                      wf1p, wf1x, z_ref, pj_ref):
    z = _relu(_dotf(x_ref[...], we1[...]) + be1[...])
    z = _dotf(z, we2[...]) + be2[...]
    z = _dotf(z, wt[...]) + bt[...]
    z_ref[...] = z
    pj_ref[...] = _dotf(pos_ref[...], wf1p[...]) + _dotf(z, wf1x[...])


def _encode_pj(x, pos, weights, tile):
    N, C = x.shape
    HP = weights[4].shape[1]
    row = lambda i: (i, 0)
    full = lambda i: (0, 0)
    return pl.pallas_call(
        _encode_pj_kernel,
        out_shape=(jax.ShapeDtypeStruct((N, HP), F32),
                   jax.ShapeDtypeStruct((N, HP), F32)),
        grid=(N // tile,),
        in_specs=[pl.BlockSpec((tile, C), row),
                  pl.BlockSpec((tile, pos.shape[1]), row)]
                 + [pl.BlockSpec(w.shape, full) for w in weights],
        out_specs=(pl.BlockSpec((tile, HP), row), pl.BlockSpec((tile, HP), row)),
        compiler_params=_cparams(),
    )(x, pos, *weights)


def _conv_body(xi, pos_i, gath_ref,
               wh1, bh1, whf, wf1p, bfold, wf2, bf2, wg1, bg1, wg2, bg2,
               *, k_nn, width=None):
    h = _relu(_dotb(xi, wh1[...]) + bh1[...])
    node = _dotb(h, whf[...]) - _dotf(pos_i, wf1p[...]) + bfold[...]
    tile = node.shape[0]
    wf2v = wf2[...]
    agg = None
    for k in range(k_nn):
        g = gath_ref[k] if width is None else gath_ref[k][:, :width]
        p = _relu(g + node).astype(BF16)
        m = jnp.dot(p, wf2v, preferred_element_type=F32)
        agg = m if agg is None else jnp.maximum(agg, m)
    agg = agg + bf2[...]
    g = _relu(_dotb(agg, wg1[...]) + bg1[...])
    return xi + _dotb(g, wg2[...]) + bg2[...]


def _conv_kernel(x_ref, pos_ref, gath_ref,
                 wh1, bh1, whf, wf1p, bfold, wf2, bf2, wg1, bg1, wg2, bg2,
                 y_ref, psum_ref, pssq_ref, *, k_nn):
    y = _conv_body(x_ref[...], pos_ref[...], gath_ref,
                   wh1, bh1, whf, wf1p, bfold, wf2, bf2, wg1, bg1, wg2, bg2,
                   k_nn=k_nn)
    y_ref[...] = y
    psum_ref[...] = jnp.sum(y, axis=0, keepdims=True)[None]
    pssq_ref[...] = jnp.sum(y * y, axis=0, keepdims=True)[None]


def _conv_bn_kernel(y_in_ref, scale, shift, pos_ref, gath_ref,
                    wh1, bh1, whf, wf1p, bfold, wf2, bf2, wg1, bg1, wg2, bg2,
                    y_ref, psum_ref, pssq_ref, *, k_nn):
    xi = _relu(y_in_ref[...] * scale[...] + shift[...])
    y = _conv_body(xi, pos_ref[...], gath_ref,
                   wh1, bh1, whf, wf1p, bfold, wf2, bf2, wg1, bg1, wg2, bg2,
                   k_nn=k_nn)
    y_ref[...] = y
    psum_ref[...] = jnp.sum(y, axis=0, keepdims=True)[None]
    pssq_ref[...] = jnp.sum(y * y, axis=0, keepdims=True)[None]


def _conv_layer(x_or_y, pos, gathered, cw, tile, *, k_nn, bn=None, off=0):
    HP = x_or_y.shape[1]
    S = gathered.shape[1]
    ntiles = S // tile
    row = lambda i: (i + off, 0)
    full = lambda i: (0, 0)
    g3 = lambda i: (0, i, 0)
    s3 = lambda i: (i, 0, 0)
    if bn is None:
        kern = functools.partial(_conv_kernel, k_nn=k_nn)
        lead_ops = (x_or_y,)
        lead_specs = [pl.BlockSpec((tile, HP), row)]
    else:
        scale, shift = bn
        kern = functools.partial(_conv_bn_kernel, k_nn=k_nn)
        lead_ops = (x_or_y, scale, shift)
        lead_specs = [pl.BlockSpec((tile, HP), row),
                      pl.BlockSpec(scale.shape, full),
                      pl.BlockSpec(shift.shape, full)]
    return pl.pallas_call(
        kern,
        out_shape=(jax.ShapeDtypeStruct((S, HP), F32),
                   jax.ShapeDtypeStruct((ntiles, 1, HP), F32),
                   jax.ShapeDtypeStruct((ntiles, 1, HP), F32)),
        grid=(ntiles,),
        in_specs=lead_specs
                 + [pl.BlockSpec((tile, pos.shape[1]), row),
                    pl.BlockSpec((k_nn, tile, HP), g3)]
                 + [pl.BlockSpec(w.shape, full) for w in cw],
        out_specs=(pl.BlockSpec((tile, HP), lambda i: (i, 0)),
                   pl.BlockSpec((1, 1, HP), s3),
                   pl.BlockSpec((1, 1, HP), s3)),
        compiler_params=_cparams(),
    )(*lead_ops, pos, gathered, *cw)


def _bn_scale_shift(psum, pssq, gamma, beta, n):
    mean = psum.sum(axis=0) / n
    var = jnp.maximum(pssq.sum(axis=0) / n - mean * mean, 0.0)
    scale = gamma * lax.rsqrt(var + 1e-5)
    shift = beta - mean * scale
    return scale, shift


def _bn_pj_kernel(y_ref, pos_ref, scale, shift, wf1p, wf1x, pj_ref):
    xn = _relu(y_ref[...] * scale[...] + shift[...])
    pj_ref[...] = _dotf(pos_ref[...], wf1p[...]) + _dotb(xn, wf1x[...])


def _bn_pj(y, pos, scale, shift, wf1p, wf1x, tile):
    N, HP = y.shape
    HO = wf1x.shape[1]
    row = lambda i: (i, 0)
    full = lambda i: (0, 0)
    return pl.pallas_call(
        _bn_pj_kernel,
        out_shape=jax.ShapeDtypeStruct((N, HO), F32),
        grid=(N // tile,),
        in_specs=[pl.BlockSpec((tile, HP), row),
                  pl.BlockSpec((tile, pos.shape[1]), row),
                  pl.BlockSpec(scale.shape, full), pl.BlockSpec(shift.shape, full),
                  pl.BlockSpec(wf1p.shape, full), pl.BlockSpec(wf1x.shape, full)],
        out_specs=pl.BlockSpec((tile, HO), row),
        compiler_params=_cparams(),
    )(y, pos, scale, shift, wf1p, wf1x)


def _bn_proj_pj_kernel(y_ref, pos_ref, scale, shift, wp, bp, wf1p, wf1x,
                       xd_ref, pj_ref):
    xn = _relu(y_ref[...] * scale[...] + shift[...])
    xd = _dotb(xn, wp[...]) + bp[...]
    xd_ref[...] = xd
    pj_ref[...] = _dotf(pos_ref[...], wf1p[...]) + _dotb(xd, wf1x[...])


def _bn_proj_pj(y, pos, scale, shift, wp, bp, wf1p, wf1x, tile):
    N, HP = y.shape
    D = wp.shape[1]
    HO = wf1x.shape[1]
    row = lambda i: (i, 0)
    full = lambda i: (0, 0)
    return pl.pallas_call(
        _bn_proj_pj_kernel,
        out_shape=(jax.ShapeDtypeStruct((N, D), F32),
                   jax.ShapeDtypeStruct((N, HO), F32)),
        grid=(N // tile,),
        in_specs=[pl.BlockSpec((tile, HP), row),
                  pl.BlockSpec((tile, pos.shape[1]), row),
                  pl.BlockSpec(scale.shape, full), pl.BlockSpec(shift.shape, full),
                  pl.BlockSpec(wp.shape, full), pl.BlockSpec(bp.shape, full),
                  pl.BlockSpec(wf1p.shape, full), pl.BlockSpec(wf1x.shape, full)],
        out_specs=(pl.BlockSpec((tile, D), row), pl.BlockSpec((tile, HO), row)),
        compiler_params=_cparams(),
    )(y, pos, scale, shift, wp, bp, wf1p, wf1x)


def _out_conv_dec_kernel(xd_ref, pos_ref, gath_ref,
                         wh1, bh1, whf, wf1p, bfold, wf2, bf2, wg1, bg1,
                         wg2, bg2, wd1, bd1, wd2, bd2, o_ref, *, k_nn, width):
    y = _conv_body(xd_ref[...], pos_ref[...], gath_ref,
                   wh1, bh1, whf, wf1p, bfold, wf2, bf2, wg1, bg1, wg2, bg2,
                   k_nn=k_nn, width=width)
    d = _relu(_dotb(y, wd1[...]) + bd1[...])
    o_ref[...] = _dotb(d, wd2[...]) + bd2[...]


def _out_conv_dec(xd, pos, gathered, cw, dec_w, tile, *, k_nn, off=0):
    wd1, bd1, wd2, bd2 = dec_w
    D = xd.shape[1]
    GH = gathered.shape[2]
    S = gathered.shape[1]
    CP = wd2.shape[1]
    ntiles = S // tile
    row = lambda i: (i + off, 0)
    full = lambda i: (0, 0)
    g3 = lambda i: (0, i, 0)
    weights = tuple(cw) + (wd1, bd1, wd2, bd2)
    kern = functools.partial(_out_conv_dec_kernel, k_nn=k_nn,
                             width=(D if D != GH else None))
    return pl.pallas_call(
        kern,
        out_shape=jax.ShapeDtypeStruct((S, CP), F32),
        grid=(ntiles,),
        in_specs=[pl.BlockSpec((tile, D), row),
                  pl.BlockSpec((tile, pos.shape[1]), row),
                  pl.BlockSpec((k_nn, tile, GH), g3)]
                 + [pl.BlockSpec(w.shape, full) for w in weights],
        out_specs=pl.BlockSpec((tile, CP), lambda i: (i, 0)),
        compiler_params=_cparams(),
    )(xd, pos, gathered, *weights)


def kernel(enc0_w, enc0_b, enc1_w, enc1_b, it_w, it_b,
           in_wh1, in_bh1, in_whf, in_wf1p, in_bfold, in_wf1x,
           in_wf2, in_bf2, in_wg1, in_bg1, in_wg2, in_bg2,
           h0_wh1, h0_bh1, h0_whf, h0_wf1p, h0_bfold, h0_wf1x,
           h0_wf2, h0_bf2, h0_wg1, h0_bg1, h0_wg2, h0_bg2,
           op_w, op_b,
           out_wh1, out_bh1, out_whf, out_wf1p, out_bfold, out_wf1x,
           out_wf2, out_bf2, out_wg1, out_bg1, out_wg2, out_bg2,
           dec0_w, dec0_b, dec1_w, dec1_b,
           bn0_g, bn0_b, bn1_g, bn1_b,
           x, pos, nbr_km):
    N = x.shape[0]
    k_nn = nbr_km.shape[0]
    tile = _row_tile(N)
    c_out = 4
    W = 64

    bf = lambda w: w.astype(BF16)

    in_cw = (bf(in_wh1), in_bh1, bf(in_whf), in_wf1p, in_bfold,
             bf(in_wf2), in_bf2, bf(in_wg1), in_bg1, bf(in_wg2), in_bg2)
    h0_cw = (bf(h0_wh1), h0_bh1, bf(h0_whf), h0_wf1p, h0_bfold,
             bf(h0_wf2), h0_bf2, bf(h0_wg1), h0_bg1, bf(h0_wg2), h0_bg2)
    out_cw = (bf(out_wh1[:W, :W]), out_bh1[:, :W], bf(out_whf[:W, :W]),
              out_wf1p[:, :W], out_bfold[:, :W],
              bf(out_wf2[:W, :W]), out_bf2[:, :W],
              bf(out_wg1[:W, :W]), out_bg1[:, :W],
              bf(out_wg2[:W, :W]), out_bg2[:, :W])
    dec_w = (bf(dec0_w[:W, :W]), dec0_b[:, :W], bf(dec1_w[:W]), dec1_b)

    C = 8 if N % (8 * tile) == 0 else (4 if N % (4 * tile) == 0 else 1)
    S = N // C
    chunk_nbr = [lax.slice_in_dim(nbr_km, c * S, (c + 1) * S, axis=1)
                 for c in range(C)]

    def conv_chunked(src, pj, cw, bn):
        ys, pss, pqs = [], [], []
        for c in range(C):
            g = jnp.take(pj, chunk_nbr[c], axis=0)
            y_c, ps, pq = _conv_layer(src, pos, g, cw, tile, k_nn=k_nn,
                                      bn=bn, off=c * (S // tile))
            ys.append(y_c), pss.append(ps), pqs.append(pq)
        y = ys[0] if C == 1 else jnp.concatenate(ys)
        return y, jnp.concatenate(pss), jnp.concatenate(pqs)

    z, pj = _encode_pj(x, pos,
                       (enc0_w, enc0_b, enc1_w, enc1_b, it_w, it_b,
                        in_wf1p, in_wf1x),
                       tile)

    y1, ps1, pq1 = conv_chunked(z, pj, in_cw, None)
    sc1, sh1 = _bn_scale_shift(ps1, pq1, bn0_g, bn0_b, N)

    pj = _bn_pj(y1, pos, sc1, sh1, h0_wf1p, bf(h0_wf1x), tile)
    y2, ps2, pq2 = conv_chunked(y1, pj, h0_cw, (sc1, sh1))
    sc2, sh2 = _bn_scale_shift(ps2, pq2, bn1_g, bn1_b, N)

    xd, pj = _bn_proj_pj(y2, pos, sc2, sh2, bf(op_w[:, :W]), op_b[:, :W],
                         out_wf1p, bf(out_wf1x[:W]), tile)

    outs = []
    for c in range(C):
        g = jnp.take(pj, chunk_nbr[c], axis=0)
        o_c = _out_conv_dec(xd, pos, g, out_cw, dec_w, tile, k_nn=k_nn,
                            off=c * (S // tile))
        outs.append(o_c[:, :c_out])
    return outs[0] if C == 1 else jnp.concatenate(outs)


# speedup vs baseline: 1.0482x; 1.0482x over previous
import functools

import jax
import jax.numpy as jnp
from jax import lax
from jax.experimental import pallas as pl
from jax.experimental.pallas import tpu as pltpu

F32 = jnp.float32
BF16 = jnp.bfloat16


def _cparams():
    return pltpu.CompilerParams(dimension_semantics=("parallel",),
                                vmem_limit_bytes=56 * 1024 * 1024)


def _row_tile(n):
    for t in (4096, 2048, 1024, 512, 256, 128, 64, 32, 16, 8):
        if n % t == 0:
            return t
    return n


def _relu(v):
    return jnp.maximum(v, 0.0)


def _dotf(a, b):
    return jnp.dot(a, b, preferred_element_type=F32)


def _dotb(a, w):
    return jnp.dot(a.astype(BF16), w, preferred_element_type=F32)


def _encode_pj_kernel(x_ref, pos_ref, we1, be1, we2, be2, wt, bt,
                      wf1p, wf1x, z_ref, pj_ref):
    z = _relu(_dotf(x_ref[...], we1[...]) + be1[...])
    z = _dotf(z, we2[...]) + be2[...]
    z = _dotf(z, wt[...]) + bt[...]
    z_ref[...] = z
    pj_ref[...] = _dotf(pos_ref[...], wf1p[...]) + _dotf(z, wf1x[...])


def _encode_pj(x, pos, weights, tile):
    N, C = x.shape
    HP = weights[4].shape[1]
    row = lambda i: (i, 0)
    full = lambda i: (0, 0)
    return pl.pallas_call(
        _encode_pj_kernel,
        out_shape=(jax.ShapeDtypeStruct((N, HP), F32),
                   jax.ShapeDtypeStruct((N, HP), F32)),
        grid=(N // tile,),
        in_specs=[pl.BlockSpec((tile, C), row),
                  pl.BlockSpec((tile, pos.shape[1]), row)]
                 + [pl.BlockSpec(w.shape, full) for w in weights],
        out_specs=(pl.BlockSpec((tile, HP), row), pl.BlockSpec((tile, HP), row)),
        compiler_params=_cparams(),
    )(x, pos, *weights)


def _conv_body(xi, pos_i, gath_ref,
               wh1, bh1, whf, wf1p, bfold, wf2, bf2, wg1, bg1, wg2, bg2,
               *, k_nn, width=None):
    h = _relu(_dotb(xi, wh1[...]) + bh1[...])
    node = _dotb(h, whf[...]) - _dotf(pos_i, wf1p[...]) + bfold[...]
    tile = node.shape[0]
    wf2v = wf2[...]
    agg = None
    for k in range(k_nn):
        g = gath_ref[k] if width is None else gath_ref[k][:, :width]
        p = _relu(g + node).astype(BF16)
        m = jnp.dot(p, wf2v, preferred_element_type=F32)
        agg = m if agg is None else jnp.maximum(agg, m)
    agg = agg + bf2[...]
    g = _relu(_dotb(agg, wg1[...]) + bg1[...])
    return xi + _dotb(g, wg2[...]) + bg2[...]


def _conv_kernel(x_ref, pos_ref, gath_ref,
                 wh1, bh1, whf, wf1p, bfold, wf2, bf2, wg1, bg1, wg2, bg2,
                 y_ref, psum_ref, pssq_ref, *, k_nn):
    y = _conv_body(x_ref[...], pos_ref[...], gath_ref,
                   wh1, bh1, whf, wf1p, bfold, wf2, bf2, wg1, bg1, wg2, bg2,
                   k_nn=k_nn)
    y_ref[...] = y
    psum_ref[...] = jnp.sum(y, axis=0, keepdims=True)[None]
    pssq_ref[...] = jnp.sum(y * y, axis=0, keepdims=True)[None]


def _conv_bn_kernel(y_in_ref, scale, shift, pos_ref, gath_ref,
                    wh1, bh1, whf, wf1p, bfold, wf2, bf2, wg1, bg1, wg2, bg2,
                    y_ref, psum_ref, pssq_ref, *, k_nn):
    xi = _relu(y_in_ref[...] * scale[...] + shift[...])
    y = _conv_body(xi, pos_ref[...], gath_ref,
                   wh1, bh1, whf, wf1p, bfold, wf2, bf2, wg1, bg1, wg2, bg2,
                   k_nn=k_nn)
    y_ref[...] = y
    psum_ref[...] = jnp.sum(y, axis=0, keepdims=True)[None]
    pssq_ref[...] = jnp.sum(y * y, axis=0, keepdims=True)[None]


def _conv_layer(x_or_y, pos, gathered, cw, tile, *, k_nn, bn=None, off=0):
    N, HP = x_or_y.shape
    S = gathered.shape[1]
    ntiles = S // tile
    row = lambda i: (i + off, 0)
    full = lambda i: (0, 0)
    g3 = lambda i: (0, i, 0)
    s3 = lambda i: (i, 0, 0)
    if bn is None:
        kern = functools.partial(_conv_kernel, k_nn=k_nn)
        lead_ops = (x_or_y,)
        lead_specs = [pl.BlockSpec((tile, HP), row)]
    else:
        scale, shift = bn
        kern = functools.partial(_conv_bn_kernel, k_nn=k_nn)
        lead_ops = (x_or_y, scale, shift)
        lead_specs = [pl.BlockSpec((tile, HP), row),
                      pl.BlockSpec(scale.shape, full),
                      pl.BlockSpec(shift.shape, full)]
    return pl.pallas_call(
        kern,
        out_shape=(jax.ShapeDtypeStruct((N, HP), F32),
                   jax.ShapeDtypeStruct((ntiles, 1, HP), F32),
                   jax.ShapeDtypeStruct((ntiles, 1, HP), F32)),
        grid=(ntiles,),
        in_specs=lead_specs
                 + [pl.BlockSpec((tile, pos.shape[1]), row),
                    pl.BlockSpec((k_nn, tile, HP), g3)]
                 + [pl.BlockSpec(w.shape, full) for w in cw],
        out_specs=(pl.BlockSpec((tile, HP), row),
                   pl.BlockSpec((1, 1, HP), s3),
                   pl.BlockSpec((1, 1, HP), s3)),
        input_output_aliases={0: 0},
        compiler_params=_cparams(),
    )(*lead_ops, pos, gathered, *cw)


def _bn_scale_shift(psum, pssq, gamma, beta, n):
    mean = psum.sum(axis=0) / n
    var = jnp.maximum(pssq.sum(axis=0) / n - mean * mean, 0.0)
    scale = gamma * lax.rsqrt(var + 1e-5)
    shift = beta - mean * scale
    return scale, shift


def _bn_pj_kernel(y_ref, pos_ref, scale, shift, wf1p, wf1x, pj_ref):
    xn = _relu(y_ref[...] * scale[...] + shift[...])
    pj_ref[...] = _dotf(pos_ref[...], wf1p[...]) + _dotb(xn, wf1x[...])


def _bn_pj(y, pos, scale, shift, wf1p, wf1x, tile):
    N, HP = y.shape
    HO = wf1x.shape[1]
    row = lambda i: (i, 0)
    full = lambda i: (0, 0)
    return pl.pallas_call(
        _bn_pj_kernel,
        out_shape=jax.ShapeDtypeStruct((N, HO), F32),
        grid=(N // tile,),
        in_specs=[pl.BlockSpec((tile, HP), row),
                  pl.BlockSpec((tile, pos.shape[1]), row),
                  pl.BlockSpec(scale.shape, full), pl.BlockSpec(shift.shape, full),
                  pl.BlockSpec(wf1p.shape, full), pl.BlockSpec(wf1x.shape, full)],
        out_specs=pl.BlockSpec((tile, HO), row),
        compiler_params=_cparams(),
    )(y, pos, scale, shift, wf1p, wf1x)


def _bn_proj_pj_kernel(y_ref, pos_ref, scale, shift, wp, bp, wf1p, wf1x,
                       xd_ref, pj_ref):
    xn = _relu(y_ref[...] * scale[...] + shift[...])
    xd = _dotb(xn, wp[...]) + bp[...]
    xd_ref[...] = xd
    pj_ref[...] = _dotf(pos_ref[...], wf1p[...]) + _dotb(xd, wf1x[...])


def _bn_proj_pj(y, pos, scale, shift, wp, bp, wf1p, wf1x, tile):
    N, HP = y.shape
    D = wp.shape[1]
    HO = wf1x.shape[1]
    row = lambda i: (i, 0)
    full = lambda i: (0, 0)
    return pl.pallas_call(
        _bn_proj_pj_kernel,
        out_shape=(jax.ShapeDtypeStruct((N, D), F32),
                   jax.ShapeDtypeStruct((N, HO), F32)),
        grid=(N // tile,),
        in_specs=[pl.BlockSpec((tile, HP), row),
                  pl.BlockSpec((tile, pos.shape[1]), row),
                  pl.BlockSpec(scale.shape, full), pl.BlockSpec(shift.shape, full),
                  pl.BlockSpec(wp.shape, full), pl.BlockSpec(bp.shape, full),
                  pl.BlockSpec(wf1p.shape, full), pl.BlockSpec(wf1x.shape, full)],
        out_specs=(pl.BlockSpec((tile, D), row), pl.BlockSpec((tile, HO), row)),
        compiler_params=_cparams(),
    )(y, pos, scale, shift, wp, bp, wf1p, wf1x)


def _out_conv_dec_kernel(xd_ref, pos_ref, gath_ref,
                         wh1, bh1, whf, wf1p, bfold, wf2, bf2, wg1, bg1,
                         wg2, bg2, wd1, bd1, wd2, bd2, o_ref, *, k_nn, width):
    y = _conv_body(xd_ref[...], pos_ref[...], gath_ref,
                   wh1, bh1, whf, wf1p, bfold, wf2, bf2, wg1, bg1, wg2, bg2,
                   k_nn=k_nn, width=width)
    d = _relu(_dotb(y, wd1[...]) + bd1[...])
    o_ref[...] = _dotb(d, wd2[...]) + bd2[...]


def _out_conv_dec(xd, pos, gathered, cw, dec_w, tile, *, k_nn, off=0):
    wd1, bd1, wd2, bd2 = dec_w
    D = xd.shape[1]
    GH = gathered.shape[2]
    S = gathered.shape[1]
    CP = wd2.shape[1]
    ntiles = S // tile
    row = lambda i: (i + off, 0)
    full = lambda i: (0, 0)
    g3 = lambda i: (0, i, 0)
    weights = tuple(cw) + (wd1, bd1, wd2, bd2)
    kern = functools.partial(_out_conv_dec_kernel, k_nn=k_nn,
                             width=(D if D != GH else None))
    return pl.pallas_call(
        kern,
        out_shape=jax.ShapeDtypeStruct((S, CP), F32),
        grid=(ntiles,),
        in_specs=[pl.BlockSpec((tile, D), row),
                  pl.BlockSpec((tile, pos.shape[1]), row),
                  pl.BlockSpec((k_nn, tile, GH), g3)]
                 + [pl.BlockSpec(w.shape, full) for w in weights],
        out_specs=pl.BlockSpec((tile, CP), lambda i: (i, 0)),
        compiler_params=_cparams(),
    )(xd, pos, gathered, *weights)


def kernel(enc0_w, enc0_b, enc1_w, enc1_b, it_w, it_b,
           in_wh1, in_bh1, in_whf, in_wf1p, in_bfold, in_wf1x,
           in_wf2, in_bf2, in_wg1, in_bg1, in_wg2, in_bg2,
           h0_wh1, h0_bh1, h0_whf, h0_wf1p, h0_bfold, h0_wf1x,
           h0_wf2, h0_bf2, h0_wg1, h0_bg1, h0_wg2, h0_bg2,
           op_w, op_b,
           out_wh1, out_bh1, out_whf, out_wf1p, out_bfold, out_wf1x,
           out_wf2, out_bf2, out_wg1, out_bg1, out_wg2, out_bg2,
           dec0_w, dec0_b, dec1_w, dec1_b,
           bn0_g, bn0_b, bn1_g, bn1_b,
           x, pos, nbr_km):
    N = x.shape[0]
    k_nn = nbr_km.shape[0]
    tile = _row_tile(N)
    c_out = 4
    W = 64

    bf = lambda w: w.astype(BF16)

    in_cw = (bf(in_wh1), in_bh1, bf(in_whf), in_wf1p, in_bfold,
             bf(in_wf2), in_bf2, bf(in_wg1), in_bg1, bf(in_wg2), in_bg2)
    h0_cw = (bf(h0_wh1), h0_bh1, bf(h0_whf), h0_wf1p, h0_bfold,
             bf(h0_wf2), h0_bf2, bf(h0_wg1), h0_bg1, bf(h0_wg2), h0_bg2)
    out_cw = (bf(out_wh1[:W, :W]), out_bh1[:, :W], bf(out_whf[:W, :W]),
              out_wf1p[:, :W], out_bfold[:, :W],
              bf(out_wf2[:W, :W]), out_bf2[:, :W],
              bf(out_wg1[:W, :W]), out_bg1[:, :W],
              bf(out_wg2[:W, :W]), out_bg2[:, :W])
    dec_w = (bf(dec0_w[:W, :W]), dec0_b[:, :W], bf(dec1_w[:W]), dec1_b)

    C = 4 if N % (4 * tile) == 0 else 1
    S = N // C
    chunk_nbr = [lax.slice_in_dim(nbr_km, c * S, (c + 1) * S, axis=1)
                 for c in range(C)]

    def conv_chunked(src, pj, cw, bn):
        pss, pqs = [], []
        buf = src
        for c in range(C):
            g = jnp.take(pj, chunk_nbr[c], axis=0)
            buf, ps, pq = _conv_layer(buf, pos, g, cw, tile, k_nn=k_nn,
                                      bn=bn, off=c * (S // tile))
            pss.append(ps), pqs.append(pq)
        return buf, jnp.concatenate(pss), jnp.concatenate(pqs)

    z, pj = _encode_pj(x, pos,
                       (enc0_w, enc0_b, enc1_w, enc1_b, it_w, it_b,
                        in_wf1p, in_wf1x),
                       tile)

    y1, ps1, pq1 = conv_chunked(z, pj, in_cw, None)
    sc1, sh1 = _bn_scale_shift(ps1, pq1, bn0_g, bn0_b, N)

    pj = _bn_pj(y1, pos, sc1, sh1, h0_wf1p, bf(h0_wf1x), tile)
    y2, ps2, pq2 = conv_chunked(y1, pj, h0_cw, (sc1, sh1))
    sc2, sh2 = _bn_scale_shift(ps2, pq2, bn1_g, bn1_b, N)

    xd, pj = _bn_proj_pj(y2, pos, sc2, sh2, bf(op_w[:, :W]), op_b[:, :W],
                         out_wf1p, bf(out_wf1x[:W]), tile)

    outs = []
    for c in range(C):
        g = jnp.take(pj, chunk_nbr[c], axis=0)
        o_c = _out_conv_dec(xd, pos, g, out_cw, dec_w, tile, k_nn=k_nn,
                            off=c * (S // tile))
        outs.append(o_c[:, :c_out])
    return outs[0] if C == 1 else jnp.concatenate(outs)
